# baseline (speedup 1.0000x reference)
import jax
import jax.numpy as jnp
from jax.experimental import pallas as pl
from jax.experimental.pallas import tpu as pltpu

N = 10000
D = 128
ROWS = 200
NBUF = 3
NSTEPS = N // ROWS


def _gcn_kernel(x_ref, w_ref, adj_hbm, adjw_hbm, o_ref, s_ref, abuf, bbuf, sems):
    i = pl.program_id(0)

    def start_copies(step, slot):
        pltpu.make_async_copy(
            adj_hbm.at[pl.ds(step * ROWS, ROWS), :], abuf.at[slot], sems.at[0, slot]
        ).start()
        pltpu.make_async_copy(
            adjw_hbm.at[pl.ds(step * ROWS, ROWS), :], bbuf.at[slot], sems.at[1, slot]
        ).start()

    @pl.when(i == 0)
    def _():
        start_copies(0, 0)
        start_copies(1, 1)
        start_copies(2, 2)
        s_ref[...] = jax.lax.dot(
            x_ref[...], w_ref[...], preferred_element_type=jnp.float32
        )

    slot = jax.lax.rem(i, NBUF)
    pltpu.make_async_copy(
        adj_hbm.at[pl.ds(i * ROWS, ROWS), :], abuf.at[slot], sems.at[0, slot]
    ).wait()
    pltpu.make_async_copy(
        adjw_hbm.at[pl.ds(i * ROWS, ROWS), :], bbuf.at[slot], sems.at[1, slot]
    ).wait()

    a = abuf[slot] + bbuf[slot]
    out = jax.lax.dot(a, s_ref[...], preferred_element_type=jnp.float32)
    norm = jnp.sqrt(jnp.sum(out * out, axis=-1, keepdims=True))
    o_ref[...] = out / jnp.maximum(norm, 1e-12)

    @pl.when(i + NBUF < NSTEPS)
    def _():
        start_copies(i + NBUF, slot)


def kernel(x, adj, adj_w, W):
    return pl.pallas_call(
        _gcn_kernel,
        grid=(NSTEPS,),
        in_specs=[
            pl.BlockSpec((N, D), lambda i: (0, 0)),
            pl.BlockSpec((D, D), lambda i: (0, 0)),
            pl.BlockSpec(memory_space=pltpu.MemorySpace.HBM),
            pl.BlockSpec(memory_space=pltpu.MemorySpace.HBM),
        ],
        out_specs=pl.BlockSpec((ROWS, D), lambda i: (i, 0)),
        out_shape=jax.ShapeDtypeStruct((N, D), jnp.float32),
        scratch_shapes=[
            pltpu.VMEM((N, D), jnp.float32),
            pltpu.VMEM((NBUF, ROWS, N), jnp.float32),
            pltpu.VMEM((NBUF, ROWS, N), jnp.float32),
            pltpu.SemaphoreType.DMA((2, NBUF)),
        ],
    )(x, W, adj, adj_w)


# speedup vs baseline: 1.0176x; 1.0176x over previous
import jax
import jax.numpy as jnp
from jax.experimental import pallas as pl
from jax.experimental.pallas import tpu as pltpu

N = 10000
D = 128
ROWS = 200


def _gcn_kernel(x_ref, w_ref, adj_ref, adjw_ref, o_ref, s_ref):
    @pl.when(pl.program_id(0) == 0)
    def _():
        s_ref[...] = jax.lax.dot(
            x_ref[...], w_ref[...], preferred_element_type=jnp.float32
        )

    a = adj_ref[...] + adjw_ref[...]
    out = jax.lax.dot(
        a[:, :2560], s_ref[:2560, :], preferred_element_type=jnp.float32
    )
    out += jax.lax.dot(
        a[:, 2560:5120], s_ref[2560:5120, :], preferred_element_type=jnp.float32
    )
    out += jax.lax.dot(
        a[:, 5120:7680], s_ref[5120:7680, :], preferred_element_type=jnp.float32
    )
    out += jax.lax.dot(
        a[:, 7680:], s_ref[7680:, :], preferred_element_type=jnp.float32
    )
    norm = jnp.sqrt(jnp.sum(out * out, axis=-1, keepdims=True))
    o_ref[...] = out / jnp.maximum(norm, 1e-12)


def kernel(x, adj, adj_w, W):
    return pl.pallas_call(
        _gcn_kernel,
        grid=(N // ROWS,),
        in_specs=[
            pl.BlockSpec((N, D), lambda i: (0, 0)),
            pl.BlockSpec((D, D), lambda i: (0, 0)),
            pl.BlockSpec((ROWS, N), lambda i: (i, 0)),
            pl.BlockSpec((ROWS, N), lambda i: (i, 0)),
        ],
        out_specs=pl.BlockSpec((ROWS, D), lambda i: (i, 0)),
        out_shape=jax.ShapeDtypeStruct((N, D), jnp.float32),
        scratch_shapes=[pltpu.VMEM((N, D), jnp.float32)],
    )(x, W, adj, adj_w)
